# baseline (speedup 1.0000x reference)
import dataclasses
import functools

import jax
import jax.numpy as jnp
from jax import lax
from jax.experimental import pallas as pl
from jax.experimental.pallas import tpu as pltpu
from jax.experimental.pallas import tpu_sc as plsc

_V = 10000
_E = 160000
_DEPTH = 5

_NC = 2
_NS = 16
_NW = _NC * _NS
_E2 = 163840
_C = 16
_EPW = _E2 // _NW
_NCHW = _EPW // _C
_VP = 10240
_ROWS = _VP // _NS

_BV = 2000
_BE = 2048


def _lrelu(x):
    return jnp.maximum(x, 0.2 * x)


def _inproj_body(x_ref, w_ref, b_ref, o_ref):
    y = jnp.dot(x_ref[...], w_ref[...], preferred_element_type=jnp.float32)
    o_ref[...] = _lrelu(y + b_ref[...])


def _tc_inproj(x, W, b):
    return pl.pallas_call(
        _inproj_body,
        grid=(_V // _BV,),
        in_specs=[pl.BlockSpec((_BV, 128), lambda i: (i, 0)),
                  pl.BlockSpec((128, 128), lambda i: (0, 0)),
                  pl.BlockSpec((1, 128), lambda i: (0, 0))],
        out_specs=pl.BlockSpec((_BV, 128), lambda i: (i, 0)),
        out_shape=jax.ShapeDtypeStruct((_V, 128), jnp.float32),
    )(x, W, b.reshape(1, 128))


def _nodeproj_body(x_ref, w_ref, b_ref, s_ref, d_ref, wn_ref):
    y = jnp.dot(x_ref[...], w_ref[...], preferred_element_type=jnp.float32)
    y = y + b_ref[...]
    s_ref[...] = y[:, :256]
    d_ref[...] = y[:, 256:512]
    wn_ref[...] = y[:, 512:]


def _tc_nodeproj(h, Wn, bn):
    return pl.pallas_call(
        _nodeproj_body,
        grid=(_V // _BV,),
        in_specs=[pl.BlockSpec((_BV, 128), lambda i: (i, 0)),
                  pl.BlockSpec((128, 640), lambda i: (0, 0)),
                  pl.BlockSpec((1, 640), lambda i: (0, 0))],
        out_specs=[pl.BlockSpec((_BV, 256), lambda i: (i, 0)),
                   pl.BlockSpec((_BV, 256), lambda i: (i, 0)),
                   pl.BlockSpec((_BV, 128), lambda i: (i, 0))],
        out_shape=[jax.ShapeDtypeStruct((_V, 256), jnp.float32),
                   jax.ShapeDtypeStruct((_V, 256), jnp.float32),
                   jax.ShapeDtypeStruct((_V, 128), jnp.float32)],
    )(h, Wn, bn.reshape(1, 640))


def _edgeproj_body(x_ref, w_ref, b_ref, o_ref):
    y = jnp.dot(x_ref[...], w_ref[...], preferred_element_type=jnp.float32)
    o_ref[...] = y + b_ref[...]


def _tc_edgeproj(ef, Wep, bep):
    return pl.pallas_call(
        _edgeproj_body,
        grid=(_E2 // _BE,),
        in_specs=[pl.BlockSpec((_BE, 16), lambda i: (i, 0)),
                  pl.BlockSpec((16, 256), lambda i: (0, 0)),
                  pl.BlockSpec((1, 256), lambda i: (0, 0))],
        out_specs=pl.BlockSpec((_BE, 256), lambda i: (i, 0)),
        out_shape=jax.ShapeDtypeStruct((_E2, 256), jnp.float32),
    )(ef, Wep, bep.reshape(1, 256))


def _densum_body(den_ref, o_ref):
    o_ref[...] = jnp.sum(den_ref[...], axis=0, keepdims=True)


def _tc_densum(DEN):
    return pl.pallas_call(
        _densum_body,
        in_specs=[pl.BlockSpec((_NW, _VP), lambda: (0, 0))],
        out_specs=pl.BlockSpec((1, _VP), lambda: (0, 0)),
        out_shape=jax.ShapeDtypeStruct((1, _VP), jnp.float32),
    )(DEN.reshape(_NW, _VP))


def _combine_body(acc_ref, den_ref, wn_ref, atom_ref, h_ref):
    num = acc_ref[0] + acc_ref[1]
    agg = num / (den_ref[...] + 1e-16)
    h_ref[...] = jnp.maximum(_lrelu(agg + wn_ref[...]) + atom_ref[...], 0.0)


def _tc_combine(ACC, den_col, wn, atom_input):
    return pl.pallas_call(
        _combine_body,
        grid=(_V // _BV,),
        in_specs=[pl.BlockSpec((_NC, _BV, 128), lambda i: (0, i, 0)),
                  pl.BlockSpec((_BV, 1), lambda i: (i, 0)),
                  pl.BlockSpec((_BV, 128), lambda i: (i, 0)),
                  pl.BlockSpec((_BV, 128), lambda i: (i, 0))],
        out_specs=pl.BlockSpec((_BV, 128), lambda i: (i, 0)),
        out_shape=jax.ShapeDtypeStruct((_V, 128), jnp.float32),
    )(ACC, den_col, wn, atom_input)


def _readout_body(nf_ref, h_ref, wo1_ref, wo2_ref, bo_ref,
                  wr1a_ref, wr1b_ref, br1_ref, wr2_ref, br2_ref,
                  o_ref, acc_ref):
    i = pl.program_id(0)
    ah = (jnp.dot(nf_ref[...], wo1_ref[...], preferred_element_type=jnp.float32)
          + jnp.dot(h_ref[...], wo2_ref[...], preferred_element_type=jnp.float32)
          + bo_ref[...])
    ah = _lrelu(ah)
    r1 = (jnp.dot(ah, wr1a_ref[...], preferred_element_type=jnp.float32)
          + jnp.dot(nf_ref[...], wr1b_ref[...], preferred_element_type=jnp.float32))
    part = jnp.sum(r1, axis=0, keepdims=True)

    @pl.when(i == 0)
    def _():
        acc_ref[...] = jnp.zeros_like(acc_ref)

    acc_ref[...] += part

    @pl.when(i == pl.num_programs(0) - 1)
    def _():
        tot = acc_ref[...] + _V * br1_ref[...]
        o_ref[...] = (jnp.dot(tot, wr2_ref[...], preferred_element_type=jnp.float32)
                      + _V * br2_ref[...])


def _tc_readout(nf, h, W_out, b_out, W_r1, b_r1, W_r2, b_r2):
    full = lambda shape: pl.BlockSpec(shape, lambda i: tuple(0 for _ in shape))
    return pl.pallas_call(
        _readout_body,
        grid=(_V // _BV,),
        in_specs=[pl.BlockSpec((_BV, 128), lambda i: (i, 0)),
                  pl.BlockSpec((_BV, 128), lambda i: (i, 0)),
                  full((128, 128)), full((128, 128)), full((1, 128)),
                  full((128, 128)), full((128, 128)), full((1, 128)),
                  full((128, 1)), full((1, 1))],
        out_specs=pl.BlockSpec((1, 1), lambda i: (0, 0)),
        out_shape=jax.ShapeDtypeStruct((1, 1), jnp.float32),
        scratch_shapes=[pltpu.VMEM((1, 128), jnp.float32)],
    )(nf, h, W_out[:128], W_out[128:], b_out.reshape(1, 128),
      W_r1[:128], W_r1[128:], b_r1.reshape(1, 128),
      W_r2, b_r2.reshape(1, 1))


def _sc_compiler_params():
    cp = pltpu.CompilerParams()
    if "needs_layout_passes" in pltpu.CompilerParams.__dataclass_fields__:
        cp = dataclasses.replace(cp, needs_layout_passes=False)
    return cp


def _sc_edge_body(S_hbm, D_hbm, EP_hbm, src_hbm, dst_hbm,
                  wdot_hbm, bdot_hbm, z_hbm, zden_hbm, acc_hbm, den_hbm,
                  src_all, dst_all, idst0, idst1,
                  srow0, srow1, drow0, drow1, eprow0, eprow1,
                  orow, denl, wv, bv, wvec, ashared,
                  semS0, semS1, semD0, semD1, semE0, semE1):
    cid = lax.axis_index("c")
    sid = lax.axis_index("s")
    wid = sid * _NC + cid
    row0 = pl.multiple_of(sid * _ROWS, 8)
    ebase = pl.multiple_of(wid * _EPW, 8)
    iota16 = lax.iota(jnp.int32, 16)
    mask0 = iota16 == 0
    pltpu.sync_copy(wdot_hbm, wv)
    pltpu.sync_copy(bdot_hbm, bv)
    pltpu.sync_copy(src_hbm.at[pl.ds(ebase, _EPW)], src_all)
    pltpu.sync_copy(dst_hbm.at[pl.ds(ebase, _EPW)], dst_all)
    pltpu.sync_copy(zden_hbm, denl)
    pltpu.sync_copy(z_hbm.at[pl.ds(row0, _ROWS)],
                    ashared.at[pl.ds(row0, _ROWS)])
    plsc.subcore_barrier()

    bufs = ((srow0, drow0, eprow0, idst0, semS0, semD0, semE0),
            (srow1, drow1, eprow1, idst1, semS1, semD1, semE1))

    def gstart(b, n):
        srow, drow, eprow, idst, ss, sd, se = bufs[b]
        loc = pl.multiple_of(n * _C, 8)
        off = pl.multiple_of(ebase + n * _C, 8)
        idst[...] = dst_all[pl.ds(loc, _C)]
        pltpu.make_async_copy(S_hbm.at[src_all.at[pl.ds(loc, _C)]],
                              srow, ss).start()
        pltpu.make_async_copy(D_hbm.at[idst], drow, sd).start()
        pltpu.make_async_copy(EP_hbm.at[pl.ds(off, _C)], eprow, se).start()

    def gwait(b, n):
        srow, drow, eprow, idst, ss, sd, se = bufs[b]
        loc = pl.multiple_of(n * _C, 8)
        off = pl.multiple_of(ebase + n * _C, 8)
        pltpu.make_async_copy(S_hbm.at[src_all.at[pl.ds(loc, _C)]],
                              srow, ss).wait()
        pltpu.make_async_copy(D_hbm.at[idst], drow, sd).wait()
        pltpu.make_async_copy(EP_hbm.at[pl.ds(off, _C)], eprow, se).wait()

    def compute(b, n):
        srow, drow, eprow, idst = bufs[b][:4]

        @plsc.parallel_loop(0, _C, 1, unroll=2)
        def _edge(i):
            sacc = None
            for j in range(8):
                sl = pl.ds(16 * j, 16)
                a = srow[i, sl] + drow[i, sl] + eprow[i, sl]
                a = jnp.maximum(a, 0.2 * a)
                t = a * wv[sl]
                sacc = t if sacc is None else sacc + t
            wexp = jnp.exp(lax.broadcast(jnp.sum(sacc), (16,)) + bv[...])
            i16 = lax.broadcast(i, (16,))
            plsc.store_scatter(wvec, [i16], wexp, mask=iota16 == i16)
            for j in range(8):
                m = (srow[i, pl.ds(128 + 16 * j, 16)]
                     + drow[i, pl.ds(128 + 16 * j, 16)]
                     + eprow[i, pl.ds(128 + 16 * j, 16)])
                m = jnp.maximum(m, 0.2 * m)
                orow[i, pl.ds(16 * j, 16)] = m * wexp

        @pl.loop(0, _C)
        def _den(i):
            i16 = lax.broadcast(i, (16,))
            wi = plsc.load_gather(wvec, [i16])
            di = plsc.load_gather(idst, [i16])
            plsc.addupdate_scatter(denl, [di], wi, mask=mask0)

        pltpu.sync_copy(orow, ashared.at[idst], add=True)

    gstart(0, 0)

    @pl.loop(0, _NCHW, step=2)
    def _pair(n):
        gstart(1, n + 1)
        gwait(0, n)
        compute(0, n)

        @pl.when(n + 2 < _NCHW)
        def _():
            gstart(0, n + 2)

        gwait(1, n + 1)
        compute(1, n + 1)

    plsc.subcore_barrier()
    pltpu.sync_copy(ashared.at[pl.ds(row0, _ROWS)],
                    acc_hbm.at[cid].at[pl.ds(row0, _ROWS)])
    pltpu.sync_copy(denl, den_hbm.at[cid].at[sid])


@functools.cache
def _sc_edge_run():
    mesh = plsc.VectorSubcoreMesh(core_axis_name="c", subcore_axis_name="s")
    return pl.kernel(
        _sc_edge_body,
        mesh=mesh,
        out_type=[jax.ShapeDtypeStruct((_NC, _VP, 128), jnp.float32),
                  jax.ShapeDtypeStruct((_NC, _NS, _VP), jnp.float32)],
        scratch_types=[
            pltpu.VMEM((_EPW,), jnp.int32),
            pltpu.VMEM((_EPW,), jnp.int32),
            pltpu.VMEM((_C,), jnp.int32),
            pltpu.VMEM((_C,), jnp.int32),
            pltpu.VMEM((_C, 256), jnp.float32),
            pltpu.VMEM((_C, 256), jnp.float32),
            pltpu.VMEM((_C, 256), jnp.float32),
            pltpu.VMEM((_C, 256), jnp.float32),
            pltpu.VMEM((_C, 256), jnp.float32),
            pltpu.VMEM((_C, 256), jnp.float32),
            pltpu.VMEM((_C, 128), jnp.float32),
            pltpu.VMEM((_VP,), jnp.float32),
            pltpu.VMEM((128,), jnp.float32),
            pltpu.VMEM((16,), jnp.float32),
            pltpu.VMEM((16,), jnp.float32),
            pltpu.VMEM_SHARED((_VP, 128), jnp.float32),
            pltpu.SemaphoreType.DMA,
            pltpu.SemaphoreType.DMA,
            pltpu.SemaphoreType.DMA,
            pltpu.SemaphoreType.DMA,
            pltpu.SemaphoreType.DMA,
            pltpu.SemaphoreType.DMA,
        ],
        compiler_params=_sc_compiler_params(),
    )


def _sc_edge_pass(S, D, EP, src3, dst3, wdot, bdot16, zrows, zden):
    return _sc_edge_run()(S, D, EP, src3, dst3, wdot, bdot16, zrows, zden)


def kernel(node_feats, edge_feats, edge_index, W_as, b_as, W_ad, b_ad,
           W_ae, b_ae, W_dot, b_dot, W_ms, b_ms, W_md, b_md, W_me, b_me,
           W_wn, b_wn, W_inp, b_inp, W_out, b_out, W_r1, b_r1, W_r2, b_r2):
    npad = _E2 - _E
    src3 = jnp.pad(edge_index[0], (0, npad))
    dst3 = jnp.pad(edge_index[1], (0, npad), constant_values=_V)
    ef2 = jnp.pad(edge_feats, ((0, npad), (0, 0)))

    Wn_all = jnp.concatenate([W_as, W_ms, W_ad, W_md, W_wn], axis=2)
    bn_all = jnp.concatenate([b_as, b_ms, b_ad, b_md, b_wn], axis=1)
    Wep_all = jnp.concatenate([W_ae, W_me], axis=2)
    bep_all = jnp.concatenate([b_ae, b_me], axis=1)
    zrows = jnp.zeros((_VP, 128), jnp.float32)
    zden = jnp.zeros((_VP,), jnp.float32)

    atom_input = _tc_inproj(node_feats, W_inp, b_inp)
    h = atom_input
    for i in range(_DEPTH):
        S, D, wn = _tc_nodeproj(h, Wn_all[i], bn_all[i])
        EP = _tc_edgeproj(ef2, Wep_all[i], bep_all[i])
        bdot16 = jnp.full((16,), b_dot[i, 0], jnp.float32)
        ACC, DEN = _sc_edge_pass(S, D, EP, src3, dst3,
                                 W_dot[i, :, 0], bdot16, zrows, zden)
        den_col = _tc_densum(DEN).reshape(_VP, 1)
        h = _tc_combine(ACC, den_col, wn, atom_input)

    return _tc_readout(node_feats, h, W_out, b_out, W_r1, b_r1, W_r2, b_r2)


# speedup vs baseline: 1.1162x; 1.1162x over previous
import dataclasses
import functools

import jax
import jax.numpy as jnp
from jax import lax
from jax.experimental import pallas as pl
from jax.experimental.pallas import tpu as pltpu
from jax.experimental.pallas import tpu_sc as plsc

_V = 10000
_E = 160000
_DEPTH = 5

_NC = 2
_NS = 16
_NW = _NC * _NS
_E2 = 163840
_C = 16
_EPW = _E2 // _NW
_NCHW = _EPW // _C
_VP = 10240
_ROWS = _VP // _NS

_BV = 2000
_BE = 2048


def _lrelu(x):
    return jnp.maximum(x, 0.2 * x)


def _inproj_body(x_ref, w_ref, b_ref, o_ref):
    y = jnp.dot(x_ref[...], w_ref[...], preferred_element_type=jnp.float32)
    o_ref[...] = _lrelu(y + b_ref[...])


def _tc_inproj(x, W, b):
    return pl.pallas_call(
        _inproj_body,
        grid=(_V // _BV,),
        in_specs=[pl.BlockSpec((_BV, 128), lambda i: (i, 0)),
                  pl.BlockSpec((128, 128), lambda i: (0, 0)),
                  pl.BlockSpec((1, 128), lambda i: (0, 0))],
        out_specs=pl.BlockSpec((_BV, 128), lambda i: (i, 0)),
        out_shape=jax.ShapeDtypeStruct((_V, 128), jnp.float32),
    )(x, W, b.reshape(1, 128))


def _nodeproj_body(x_ref, w_ref, b_ref, s_ref, d_ref, wn_ref):
    y = jnp.dot(x_ref[...], w_ref[...], preferred_element_type=jnp.float32)
    y = y + b_ref[...]
    s_ref[...] = y[:, :256]
    d_ref[...] = y[:, 256:512]
    wn_ref[...] = y[:, 512:]


def _tc_nodeproj(h, Wn, bn):
    return pl.pallas_call(
        _nodeproj_body,
        grid=(_V // _BV,),
        in_specs=[pl.BlockSpec((_BV, 128), lambda i: (i, 0)),
                  pl.BlockSpec((128, 640), lambda i: (0, 0)),
                  pl.BlockSpec((1, 640), lambda i: (0, 0))],
        out_specs=[pl.BlockSpec((_BV, 256), lambda i: (i, 0)),
                   pl.BlockSpec((_BV, 256), lambda i: (i, 0)),
                   pl.BlockSpec((_BV, 128), lambda i: (i, 0))],
        out_shape=[jax.ShapeDtypeStruct((_V, 256), jnp.float32),
                   jax.ShapeDtypeStruct((_V, 256), jnp.float32),
                   jax.ShapeDtypeStruct((_V, 128), jnp.float32)],
    )(h, Wn, bn.reshape(1, 640))


def _edgeproj_body(x_ref, w_ref, b_ref, o_ref):
    y = jnp.dot(x_ref[...], w_ref[...], preferred_element_type=jnp.float32)
    o_ref[...] = y + b_ref[...]


def _tc_edgeproj(ef, Wep, bep):
    return pl.pallas_call(
        _edgeproj_body,
        grid=(_E2 // _BE,),
        in_specs=[pl.BlockSpec((_BE, 16), lambda i: (i, 0)),
                  pl.BlockSpec((16, 256), lambda i: (0, 0)),
                  pl.BlockSpec((1, 256), lambda i: (0, 0))],
        out_specs=pl.BlockSpec((_BE, 256), lambda i: (i, 0)),
        out_shape=jax.ShapeDtypeStruct((_E2, 256), jnp.float32),
    )(ef, Wep, bep.reshape(1, 256))


def _densum_body(den_ref, o_ref):
    o_ref[...] = jnp.sum(den_ref[...], axis=0, keepdims=True)


def _tc_densum(DEN):
    return pl.pallas_call(
        _densum_body,
        in_specs=[pl.BlockSpec((_NW, _VP), lambda: (0, 0))],
        out_specs=pl.BlockSpec((1, _VP), lambda: (0, 0)),
        out_shape=jax.ShapeDtypeStruct((1, _VP), jnp.float32),
    )(DEN.reshape(_NW, _VP))


def _combine_body(acc_ref, den_ref, wn_ref, atom_ref, h_ref):
    num = acc_ref[0] + acc_ref[1]
    agg = num / (den_ref[...] + 1e-16)
    h_ref[...] = jnp.maximum(_lrelu(agg + wn_ref[...]) + atom_ref[...], 0.0)


def _tc_combine(ACC, den_col, wn, atom_input):
    return pl.pallas_call(
        _combine_body,
        grid=(_V // _BV,),
        in_specs=[pl.BlockSpec((_NC, _BV, 128), lambda i: (0, i, 0)),
                  pl.BlockSpec((_BV, 1), lambda i: (i, 0)),
                  pl.BlockSpec((_BV, 128), lambda i: (i, 0)),
                  pl.BlockSpec((_BV, 128), lambda i: (i, 0))],
        out_specs=pl.BlockSpec((_BV, 128), lambda i: (i, 0)),
        out_shape=jax.ShapeDtypeStruct((_V, 128), jnp.float32),
    )(ACC, den_col, wn, atom_input)


def _readout_body(nf_ref, h_ref, wo1_ref, wo2_ref, bo_ref,
                  wr1a_ref, wr1b_ref, br1_ref, wr2_ref, br2_ref,
                  o_ref, acc_ref):
    i = pl.program_id(0)
    ah = (jnp.dot(nf_ref[...], wo1_ref[...], preferred_element_type=jnp.float32)
          + jnp.dot(h_ref[...], wo2_ref[...], preferred_element_type=jnp.float32)
          + bo_ref[...])
    ah = _lrelu(ah)
    r1 = (jnp.dot(ah, wr1a_ref[...], preferred_element_type=jnp.float32)
          + jnp.dot(nf_ref[...], wr1b_ref[...], preferred_element_type=jnp.float32))
    part = jnp.sum(r1, axis=0, keepdims=True)

    @pl.when(i == 0)
    def _():
        acc_ref[...] = jnp.zeros_like(acc_ref)

    acc_ref[...] += part

    @pl.when(i == pl.num_programs(0) - 1)
    def _():
        tot = acc_ref[...] + _V * br1_ref[...]
        o_ref[...] = (jnp.dot(tot, wr2_ref[...], preferred_element_type=jnp.float32)
                      + _V * br2_ref[...])


def _tc_readout(nf, h, W_out, b_out, W_r1, b_r1, W_r2, b_r2):
    full = lambda shape: pl.BlockSpec(shape, lambda i: tuple(0 for _ in shape))
    return pl.pallas_call(
        _readout_body,
        grid=(_V // _BV,),
        in_specs=[pl.BlockSpec((_BV, 128), lambda i: (i, 0)),
                  pl.BlockSpec((_BV, 128), lambda i: (i, 0)),
                  full((128, 128)), full((128, 128)), full((1, 128)),
                  full((128, 128)), full((128, 128)), full((1, 128)),
                  full((128, 1)), full((1, 1))],
        out_specs=pl.BlockSpec((1, 1), lambda i: (0, 0)),
        out_shape=jax.ShapeDtypeStruct((1, 1), jnp.float32),
        scratch_shapes=[pltpu.VMEM((1, 128), jnp.float32)],
    )(nf, h, W_out[:128], W_out[128:], b_out.reshape(1, 128),
      W_r1[:128], W_r1[128:], b_r1.reshape(1, 128),
      W_r2, b_r2.reshape(1, 1))


def _sc_compiler_params():
    cp = pltpu.CompilerParams()
    if "needs_layout_passes" in pltpu.CompilerParams.__dataclass_fields__:
        cp = dataclasses.replace(cp, needs_layout_passes=False)
    return cp


def _sc_edge_body(S_hbm, D_hbm, EP_hbm, src_hbm, dst_hbm,
                  wdot_hbm, bdot_hbm, z_hbm, zden_hbm, acc_hbm, den_hbm,
                  src_all, dst_all, idst0, idst1,
                  srow0, srow1, drow0, drow1, eprow0, eprow1,
                  orow, denl, wv, bv, wvec, ashared,
                  semS0, semS1, semD0, semD1, semE0, semE1):
    cid = lax.axis_index("c")
    sid = lax.axis_index("s")
    wid = sid * _NC + cid
    row0 = pl.multiple_of(sid * _ROWS, 8)
    ebase = pl.multiple_of(wid * _EPW, 8)
    iota16 = lax.iota(jnp.int32, 16)
    mask0 = iota16 == 0
    pltpu.sync_copy(wdot_hbm, wv)
    pltpu.sync_copy(bdot_hbm, bv)
    pltpu.sync_copy(src_hbm.at[pl.ds(ebase, _EPW)], src_all)
    pltpu.sync_copy(dst_hbm.at[pl.ds(ebase, _EPW)], dst_all)
    pltpu.sync_copy(zden_hbm, denl)
    pltpu.sync_copy(z_hbm.at[pl.ds(row0, _ROWS)],
                    ashared.at[pl.ds(row0, _ROWS)])
    plsc.subcore_barrier()

    bufs = ((srow0, drow0, eprow0, idst0, semS0, semD0, semE0),
            (srow1, drow1, eprow1, idst1, semS1, semD1, semE1))

    def gstart(b, n):
        srow, drow, eprow, idst, ss, sd, se = bufs[b]
        loc = pl.multiple_of(n * _C, 8)
        off = pl.multiple_of(ebase + n * _C, 8)
        idst[...] = dst_all[pl.ds(loc, _C)]
        pltpu.make_async_copy(S_hbm.at[src_all.at[pl.ds(loc, _C)]],
                              srow, ss).start()
        pltpu.make_async_copy(D_hbm.at[idst], drow, sd).start()
        pltpu.make_async_copy(EP_hbm.at[pl.ds(off, _C)], eprow, se).start()

    def gwait(b, n):
        srow, drow, eprow, idst, ss, sd, se = bufs[b]
        loc = pl.multiple_of(n * _C, 8)
        off = pl.multiple_of(ebase + n * _C, 8)
        pltpu.make_async_copy(S_hbm.at[src_all.at[pl.ds(loc, _C)]],
                              srow, ss).wait()
        pltpu.make_async_copy(D_hbm.at[idst], drow, sd).wait()
        pltpu.make_async_copy(EP_hbm.at[pl.ds(off, _C)], eprow, se).wait()

    def compute(b, n):
        srow, drow, eprow, idst = bufs[b][:4]

        @plsc.parallel_loop(0, _C, 1, unroll=8)
        def _edge(i):
            sacc = None
            for j in range(8):
                sl = pl.ds(16 * j, 16)
                a = srow[i, sl] + drow[i, sl] + eprow[i, sl]
                a = jnp.maximum(a, 0.2 * a)
                t = a * wv[sl]
                sacc = t if sacc is None else sacc + t
            wexp = jnp.exp(lax.broadcast(jnp.sum(sacc), (16,)) + bv[...])
            i16 = lax.broadcast(i, (16,))
            plsc.store_scatter(wvec, [i16], wexp, mask=iota16 == i16)
            for j in range(8):
                m = (srow[i, pl.ds(128 + 16 * j, 16)]
                     + drow[i, pl.ds(128 + 16 * j, 16)]
                     + eprow[i, pl.ds(128 + 16 * j, 16)])
                m = jnp.maximum(m, 0.2 * m)
                orow[i, pl.ds(16 * j, 16)] = m * wexp

        @pl.loop(0, _C)
        def _den(i):
            i16 = lax.broadcast(i, (16,))
            wi = plsc.load_gather(wvec, [i16])
            di = plsc.load_gather(idst, [i16])
            plsc.addupdate_scatter(denl, [di], wi, mask=mask0)

        pltpu.sync_copy(orow, ashared.at[idst], add=True)

    gstart(0, 0)

    @pl.loop(0, _NCHW, step=2)
    def _pair(n):
        gstart(1, n + 1)
        gwait(0, n)
        compute(0, n)

        @pl.when(n + 2 < _NCHW)
        def _():
            gstart(0, n + 2)

        gwait(1, n + 1)
        compute(1, n + 1)

    plsc.subcore_barrier()
    pltpu.sync_copy(ashared.at[pl.ds(row0, _ROWS)],
                    acc_hbm.at[cid].at[pl.ds(row0, _ROWS)])
    pltpu.sync_copy(denl, den_hbm.at[cid].at[sid])


@functools.cache
def _sc_edge_run():
    mesh = plsc.VectorSubcoreMesh(core_axis_name="c", subcore_axis_name="s")
    return pl.kernel(
        _sc_edge_body,
        mesh=mesh,
        out_type=[jax.ShapeDtypeStruct((_NC, _VP, 128), jnp.float32),
                  jax.ShapeDtypeStruct((_NC, _NS, _VP), jnp.float32)],
        scratch_types=[
            pltpu.VMEM((_EPW,), jnp.int32),
            pltpu.VMEM((_EPW,), jnp.int32),
            pltpu.VMEM((_C,), jnp.int32),
            pltpu.VMEM((_C,), jnp.int32),
            pltpu.VMEM((_C, 256), jnp.float32),
            pltpu.VMEM((_C, 256), jnp.float32),
            pltpu.VMEM((_C, 256), jnp.float32),
            pltpu.VMEM((_C, 256), jnp.float32),
            pltpu.VMEM((_C, 256), jnp.float32),
            pltpu.VMEM((_C, 256), jnp.float32),
            pltpu.VMEM((_C, 128), jnp.float32),
            pltpu.VMEM((_VP,), jnp.float32),
            pltpu.VMEM((128,), jnp.float32),
            pltpu.VMEM((16,), jnp.float32),
            pltpu.VMEM((16,), jnp.float32),
            pltpu.VMEM_SHARED((_VP, 128), jnp.float32),
            pltpu.SemaphoreType.DMA,
            pltpu.SemaphoreType.DMA,
            pltpu.SemaphoreType.DMA,
            pltpu.SemaphoreType.DMA,
            pltpu.SemaphoreType.DMA,
            pltpu.SemaphoreType.DMA,
        ],
        compiler_params=_sc_compiler_params(),
    )


def _sc_edge_pass(S, D, EP, src3, dst3, wdot, bdot16, zrows, zden):
    return _sc_edge_run()(S, D, EP, src3, dst3, wdot, bdot16, zrows, zden)


def kernel(node_feats, edge_feats, edge_index, W_as, b_as, W_ad, b_ad,
           W_ae, b_ae, W_dot, b_dot, W_ms, b_ms, W_md, b_md, W_me, b_me,
           W_wn, b_wn, W_inp, b_inp, W_out, b_out, W_r1, b_r1, W_r2, b_r2):
    npad = _E2 - _E
    src3 = jnp.pad(edge_index[0], (0, npad))
    dst3 = jnp.pad(edge_index[1], (0, npad), constant_values=_V)
    ef2 = jnp.pad(edge_feats, ((0, npad), (0, 0)))

    Wn_all = jnp.concatenate([W_as, W_ms, W_ad, W_md, W_wn], axis=2)
    bn_all = jnp.concatenate([b_as, b_ms, b_ad, b_md, b_wn], axis=1)
    Wep_all = jnp.concatenate([W_ae, W_me], axis=2)
    bep_all = jnp.concatenate([b_ae, b_me], axis=1)
    zrows = jnp.zeros((_VP, 128), jnp.float32)
    zden = jnp.zeros((_VP,), jnp.float32)

    atom_input = _tc_inproj(node_feats, W_inp, b_inp)
    h = atom_input
    for i in range(_DEPTH):
        S, D, wn = _tc_nodeproj(h, Wn_all[i], bn_all[i])
        EP = _tc_edgeproj(ef2, Wep_all[i], bep_all[i])
        bdot16 = jnp.full((16,), b_dot[i, 0], jnp.float32)
        ACC, DEN = _sc_edge_pass(S, D, EP, src3, dst3,
                                 W_dot[i, :, 0], bdot16, zrows, zden)
        den_col = _tc_densum(DEN).reshape(_VP, 1)
        h = _tc_combine(ACC, den_col, wn, atom_input)

    return _tc_readout(node_feats, h, W_out, b_out, W_r1, b_r1, W_r2, b_r2)
